# baseline (speedup 1.0000x reference)
import jax
import jax.numpy as jnp
from jax import lax
from jax.experimental import pallas as pl
from jax.experimental.pallas import tpu as pltpu
from jax.experimental.pallas import tpu_sc as plsc

_VOCAB = 100000
_NC = 2
_NS = 16
_NW = _NC * _NS
_VP = 100352
_SLICE = _VP // _NS

_COLS = 16384
_P_ROWS = 50
_C_ROWS = 200
_P_CB = 1024
_C_CB = 4096
_P_BPR = _COLS // _P_CB
_C_BPR = _COLS // _C_CB
_TASKS_PER_W = 25


def _sc_hist_body(preds_hbm, capt_hbm, pred_out, capt_out,
                  pred_acc, capt_acc,
                  pst0, pst1, cst0, cst1, ones_buf, zbuf,
                  sem0, sem1, ssem0, ssem1):
  c = lax.axis_index("c")
  s = lax.axis_index("s")
  w = c * _NS + s
  t0 = w * _TASKS_PER_W

  def fill(buf, n, value):
    def body(i, carry):
      buf[pl.ds(i * 16, 16)] = jnp.full((16,), value, jnp.int32)
      return carry
    lax.fori_loop(0, n // 16, body, 0)

  fill(zbuf, _SLICE, 0)
  fill(ones_buf, _C_CB, 1)

  pltpu.sync_copy(zbuf, pred_acc.at[pl.ds(s * _SLICE, _SLICE)])
  pltpu.sync_copy(zbuf, capt_acc.at[pl.ds(s * _SLICE, _SLICE)])
  plsc.subcore_barrier()

  sems = (sem0, sem1)
  ssems = (ssem0, ssem1)

  def scatter_input(hbm, acc, bufs, blocks_per_row, cb):
    def load(k, which):
      t = t0 + k
      row = t // blocks_per_row
      col = (t % blocks_per_row) * cb
      return pltpu.async_copy(hbm.at[row, pl.ds(col, cb)], bufs[which],
                              sems[which])

    cps = [None] * _TASKS_PER_W
    scat = [None] * _TASKS_PER_W
    cps[0] = load(0, 0)
    for k in range(_TASKS_PER_W):
      cps[k].wait()
      scat[k] = pltpu.async_copy(ones_buf.at[pl.ds(0, cb)],
                                 acc.at[bufs[k % 2]], ssems[k % 2],
                                 add=True)
      if k >= 1:
        scat[k - 1].wait()
      if k + 1 < _TASKS_PER_W:
        cps[k + 1] = load(k + 1, (k + 1) % 2)
    scat[_TASKS_PER_W - 1].wait()

  scatter_input(preds_hbm, pred_acc, (pst0, pst1), _P_BPR, _P_CB)
  scatter_input(capt_hbm, capt_acc, (cst0, cst1), _C_BPR, _C_CB)
  plsc.subcore_barrier()

  pltpu.sync_copy(pred_acc.at[pl.ds(s * _SLICE, _SLICE)],
                  pred_out.at[c, pl.ds(s * _SLICE, _SLICE)])
  pltpu.sync_copy(capt_acc.at[pl.ds(s * _SLICE, _SLICE)],
                  capt_out.at[c, pl.ds(s * _SLICE, _SLICE)])


_sc_hist = pl.kernel(
    _sc_hist_body,
    out_type=(
        jax.ShapeDtypeStruct((_NC, _VP), jnp.int32),
        jax.ShapeDtypeStruct((_NC, _VP), jnp.int32),
    ),
    mesh=plsc.VectorSubcoreMesh(core_axis_name="c", subcore_axis_name="s"),
    scratch_types=(
        pltpu.VMEM_SHARED((_VP,), jnp.int32),
        pltpu.VMEM_SHARED((_VP,), jnp.int32),
        pltpu.VMEM((_P_CB,), jnp.int32),
        pltpu.VMEM((_P_CB,), jnp.int32),
        pltpu.VMEM((_C_CB,), jnp.int32),
        pltpu.VMEM((_C_CB,), jnp.int32),
        pltpu.VMEM((_C_CB,), jnp.int32),
        pltpu.VMEM((_SLICE,), jnp.int32),
        pltpu.SemaphoreType.DMA,
        pltpu.SemaphoreType.DMA,
        pltpu.SemaphoreType.DMA,
        pltpu.SemaphoreType.DMA,
    ),
)


def _tc_merge_body(ph_ref, ch_ref, out_ref):
  n_pred = jnp.sum((ph_ref[0] + ph_ref[1]) > 0).astype(jnp.float32)
  n_capt = jnp.sum((ch_ref[0] + ch_ref[1]) > 0).astype(jnp.float32)
  out_ref[0, 0] = jnp.where(
      n_capt > 0, n_pred / jnp.maximum(n_capt, 1.0), jnp.float32(0.0))


@jax.jit
def kernel(preds, captions):
  ph, ch = _sc_hist(preds.T, captions.T)
  ratio = pl.pallas_call(
      _tc_merge_body,
      out_shape=jax.ShapeDtypeStruct((1, 1), jnp.float32),
      in_specs=[
          pl.BlockSpec(memory_space=pltpu.VMEM),
          pl.BlockSpec(memory_space=pltpu.VMEM),
      ],
      out_specs=pl.BlockSpec(memory_space=pltpu.SMEM),
  )(ph, ch)
  return ratio[0, 0]


# speedup vs baseline: 1.0304x; 1.0304x over previous
import jax
import jax.numpy as jnp
from jax import lax
from jax.experimental import pallas as pl
from jax.experimental.pallas import tpu as pltpu
from jax.experimental.pallas import tpu_sc as plsc

_VOCAB = 100000
_NC = 2
_NS = 16
_NW = _NC * _NS
_VP = 100352
_SLICE = _VP // _NS

_COLS = 16384
_P_ROWS = 50
_C_ROWS = 200
_P_CB = 1024
_C_CB = 4096
_P_BPR = _COLS // _P_CB
_C_BPR = _COLS // _C_CB
_TASKS_PER_W = 25


def _sc_hist_body(preds_hbm, capt_hbm, pred_out, capt_out,
                  pred_acc, capt_acc,
                  pst0, pst1, cst0, cst1, ones_buf, zbuf,
                  sem0, sem1):
  c = lax.axis_index("c")
  s = lax.axis_index("s")
  w = c * _NS + s
  t0 = w * _TASKS_PER_W

  sems = (sem0, sem1)

  def load_from(hbm, buf, sem, blocks_per_row, cb, k):
    t = t0 + k
    row = t // blocks_per_row
    col = (t % blocks_per_row) * cb
    return pltpu.async_copy(hbm.at[row, pl.ds(col, cb)], buf, sem)

  first_p = load_from(preds_hbm, pst0, sem0, _P_BPR, _P_CB, 0)

  def fill(buf, n, value):
    def body(i, carry):
      buf[pl.ds(i * 16, 16)] = jnp.full((16,), value, jnp.int32)
      return carry
    lax.fori_loop(0, n // 16, body, 0)

  fill(zbuf, _SLICE, 0)
  fill(ones_buf, _C_CB, 1)

  pltpu.sync_copy(zbuf, pred_acc.at[pl.ds(s * _SLICE, _SLICE)])
  pltpu.sync_copy(zbuf, capt_acc.at[pl.ds(s * _SLICE, _SLICE)])
  plsc.subcore_barrier()

  def scatter_input(hbm, acc, bufs, blocks_per_row, cb, first_cp=None):
    def load(k, which):
      return load_from(hbm, bufs[which], sems[which], blocks_per_row, cb, k)

    cps = [None] * _TASKS_PER_W
    cps[0] = first_cp if first_cp is not None else load(0, 0)
    for k in range(_TASKS_PER_W):
      if k + 1 < _TASKS_PER_W:
        cps[k + 1] = load(k + 1, (k + 1) % 2)
      cps[k].wait()
      pltpu.sync_copy(ones_buf.at[pl.ds(0, cb)], acc.at[bufs[k % 2]],
                      add=True)

  scatter_input(preds_hbm, pred_acc, (pst0, pst1), _P_BPR, _P_CB,
                first_cp=first_p)
  scatter_input(capt_hbm, capt_acc, (cst0, cst1), _C_BPR, _C_CB)
  plsc.subcore_barrier()

  pltpu.sync_copy(pred_acc.at[pl.ds(s * _SLICE, _SLICE)],
                  pred_out.at[c, pl.ds(s * _SLICE, _SLICE)])
  pltpu.sync_copy(capt_acc.at[pl.ds(s * _SLICE, _SLICE)],
                  capt_out.at[c, pl.ds(s * _SLICE, _SLICE)])


_sc_hist = pl.kernel(
    _sc_hist_body,
    out_type=(
        jax.ShapeDtypeStruct((_NC, _VP), jnp.int32),
        jax.ShapeDtypeStruct((_NC, _VP), jnp.int32),
    ),
    mesh=plsc.VectorSubcoreMesh(core_axis_name="c", subcore_axis_name="s"),
    scratch_types=(
        pltpu.VMEM_SHARED((_VP,), jnp.int32),
        pltpu.VMEM_SHARED((_VP,), jnp.int32),
        pltpu.VMEM((_P_CB,), jnp.int32),
        pltpu.VMEM((_P_CB,), jnp.int32),
        pltpu.VMEM((_C_CB,), jnp.int32),
        pltpu.VMEM((_C_CB,), jnp.int32),
        pltpu.VMEM((_C_CB,), jnp.int32),
        pltpu.VMEM((_SLICE,), jnp.int32),
        pltpu.SemaphoreType.DMA,
        pltpu.SemaphoreType.DMA,
    ),
)


def _tc_merge_body(ph_ref, ch_ref, out_ref):
  n_pred = jnp.sum((ph_ref[0] + ph_ref[1]) > 0).astype(jnp.float32)
  n_capt = jnp.sum((ch_ref[0] + ch_ref[1]) > 0).astype(jnp.float32)
  out_ref[0, 0] = jnp.where(
      n_capt > 0, n_pred / jnp.maximum(n_capt, 1.0), jnp.float32(0.0))


@jax.jit
def kernel(preds, captions):
  ph, ch = _sc_hist(preds.T, captions.T)
  ratio = pl.pallas_call(
      _tc_merge_body,
      out_shape=jax.ShapeDtypeStruct((1, 1), jnp.float32),
      in_specs=[
          pl.BlockSpec(memory_space=pltpu.VMEM),
          pl.BlockSpec(memory_space=pltpu.VMEM),
      ],
      out_specs=pl.BlockSpec(memory_space=pltpu.SMEM),
  )(ph, ch)
  return ratio[0, 0]


# speedup vs baseline: 1.0421x; 1.0114x over previous
import jax
import jax.numpy as jnp
from jax import lax
from jax.experimental import pallas as pl
from jax.experimental.pallas import tpu as pltpu
from jax.experimental.pallas import tpu_sc as plsc

_VOCAB = 100000
_NC = 2
_NS = 16
_NW = _NC * _NS
_VP = 100352
_SLICE = _VP // _NS

_COLS = 16384
_P_ROWS = 50
_C_ROWS = 200
_P_CB = 1024
_C_CB = 4096
_P_BPR = _COLS // _P_CB
_C_BPR = _COLS // _C_CB
_TASKS_PER_W = 25


def _sc_hist_body(preds_hbm, capt_hbm, pred_out, capt_out,
                  pred_acc, capt_acc,
                  pst0, pst1, cst0, cst1, ones_buf, zbuf,
                  sem0, sem1, csem0, csem1):
  c = lax.axis_index("c")
  s = lax.axis_index("s")
  w = c * _NS + s
  t0 = w * _TASKS_PER_W

  def load_from(hbm, buf, sem, blocks_per_row, cb, k):
    t = t0 + k
    row = t // blocks_per_row
    col = (t % blocks_per_row) * cb
    return pltpu.async_copy(hbm.at[row, pl.ds(col, cb)], buf, sem)

  first_p = load_from(preds_hbm, pst0, sem0, _P_BPR, _P_CB, 0)
  first_c = load_from(capt_hbm, cst0, csem0, _C_BPR, _C_CB, 0)

  def fill(buf, n, value):
    def body(i, carry):
      buf[pl.ds(i * 16, 16)] = jnp.full((16,), value, jnp.int32)
      return carry
    lax.fori_loop(0, n // 16, body, 0)

  fill(zbuf, _SLICE, 0)
  fill(ones_buf, _C_CB, 1)

  pltpu.sync_copy(zbuf, pred_acc.at[pl.ds(s * _SLICE, _SLICE)])
  pltpu.sync_copy(zbuf, capt_acc.at[pl.ds(s * _SLICE, _SLICE)])
  plsc.subcore_barrier()

  def scatter_input(hbm, acc, bufs, lsems, blocks_per_row, cb,
                    first_cp=None):
    def load(k, which):
      return load_from(hbm, bufs[which], lsems[which], blocks_per_row, cb, k)

    cps = [None] * _TASKS_PER_W
    cps[0] = first_cp if first_cp is not None else load(0, 0)
    for k in range(_TASKS_PER_W):
      if k + 1 < _TASKS_PER_W:
        cps[k + 1] = load(k + 1, (k + 1) % 2)
      cps[k].wait()
      pltpu.sync_copy(ones_buf.at[pl.ds(0, cb)], acc.at[bufs[k % 2]],
                      add=True)

  scatter_input(preds_hbm, pred_acc, (pst0, pst1), (sem0, sem1),
                _P_BPR, _P_CB, first_cp=first_p)
  scatter_input(capt_hbm, capt_acc, (cst0, cst1), (csem0, csem1),
                _C_BPR, _C_CB, first_cp=first_c)
  plsc.subcore_barrier()

  pltpu.sync_copy(pred_acc.at[pl.ds(s * _SLICE, _SLICE)],
                  pred_out.at[c, pl.ds(s * _SLICE, _SLICE)])
  pltpu.sync_copy(capt_acc.at[pl.ds(s * _SLICE, _SLICE)],
                  capt_out.at[c, pl.ds(s * _SLICE, _SLICE)])


_sc_hist = pl.kernel(
    _sc_hist_body,
    out_type=(
        jax.ShapeDtypeStruct((_NC, _VP), jnp.int32),
        jax.ShapeDtypeStruct((_NC, _VP), jnp.int32),
    ),
    mesh=plsc.VectorSubcoreMesh(core_axis_name="c", subcore_axis_name="s"),
    scratch_types=(
        pltpu.VMEM_SHARED((_VP,), jnp.int32),
        pltpu.VMEM_SHARED((_VP,), jnp.int32),
        pltpu.VMEM((_P_CB,), jnp.int32),
        pltpu.VMEM((_P_CB,), jnp.int32),
        pltpu.VMEM((_C_CB,), jnp.int32),
        pltpu.VMEM((_C_CB,), jnp.int32),
        pltpu.VMEM((_C_CB,), jnp.int32),
        pltpu.VMEM((_SLICE,), jnp.int32),
        pltpu.SemaphoreType.DMA,
        pltpu.SemaphoreType.DMA,
        pltpu.SemaphoreType.DMA,
        pltpu.SemaphoreType.DMA,
    ),
)


def _tc_merge_body(ph_ref, ch_ref, out_ref):
  n_pred = jnp.sum((ph_ref[0] + ph_ref[1]) > 0).astype(jnp.float32)
  n_capt = jnp.sum((ch_ref[0] + ch_ref[1]) > 0).astype(jnp.float32)
  out_ref[0, 0] = jnp.where(
      n_capt > 0, n_pred / jnp.maximum(n_capt, 1.0), jnp.float32(0.0))


@jax.jit
def kernel(preds, captions):
  ph, ch = _sc_hist(preds.T, captions.T)
  ratio = pl.pallas_call(
      _tc_merge_body,
      out_shape=jax.ShapeDtypeStruct((1, 1), jnp.float32),
      in_specs=[
          pl.BlockSpec(memory_space=pltpu.VMEM),
          pl.BlockSpec(memory_space=pltpu.VMEM),
      ],
      out_specs=pl.BlockSpec(memory_space=pltpu.SMEM),
  )(ph, ch)
  return ratio[0, 0]
